# Initial kernel scaffold; baseline (speedup 1.0000x reference)
import jax
import jax.numpy as jnp
from jax.experimental import pallas as pl


def kernel(items, user_triple_set, user_triple_set_e, item_triple_set, item_triple_set_i, entity_table, relation_table, weight_table, att_w1, att_w2, att_w3, atta_w1, atta_w2, atta_w3):
    raise NotImplementedError("write your pallas kernel here")


# speedup vs baseline: 1.7215x; 1.7215x over previous
import functools

import jax
import jax.numpy as jnp
from jax import lax
from jax.experimental import pallas as pl
from jax.experimental.pallas import tpu as pltpu
from jax.experimental.pallas import tpu_sc as plsc

_CH = 640


def _make_sc_gather(n_ent, n_rel, n_item, dim):
    info = plsc.get_sparse_core_info()
    _NC = info.num_cores
    _NS = info.num_subcores
    _NW = _NC * _NS
    e_pw = n_ent // _NW
    r_pw = n_rel // _NW
    i_pw = n_item // _NW
    mesh = plsc.VectorSubcoreMesh(core_axis_name="c", subcore_axis_name="s")
    f32 = jnp.float32

    @functools.partial(
        pl.kernel,
        mesh=mesh,
        compiler_params=pltpu.CompilerParams(use_tc_tiling_on_sc=False),
        out_type=[
            jax.ShapeDtypeStruct((n_ent, dim), f32),
            jax.ShapeDtypeStruct((n_rel, dim), f32),
            jax.ShapeDtypeStruct((n_rel, dim), f32),
            jax.ShapeDtypeStruct((n_item, dim), f32),
        ],
        scratch_types=[
            pltpu.VMEM((e_pw,), jnp.int32),
            pltpu.VMEM((_CH, dim), f32),
            pltpu.SemaphoreType.DMA,
        ],
    )
    def gather_kernel(ent_t, rel_t, wgt_t, ent_i, rel_i, wgt_i, item_i,
                      ent_o, rel_o, wgt_o, item_o, idx_v, rows_v, gsem):
        wid = lax.axis_index("s") * _NC + lax.axis_index("c")

        def run(table, ihbm, ohbm, n_per_w):
            base = wid * n_per_w
            pltpu.sync_copy(ihbm.at[pl.ds(base, n_per_w)],
                            idx_v.at[pl.ds(0, n_per_w)])

            def body(c, carry):
                off = c * _CH
                pltpu.async_copy(table.at[idx_v.at[pl.ds(off, _CH)]],
                                 rows_v, gsem).wait()
                pltpu.sync_copy(rows_v, ohbm.at[pl.ds(base + off, _CH)])
                return carry

            lax.fori_loop(0, n_per_w // _CH, body, 0)

        run(ent_t, ent_i, ent_o, e_pw)
        run(rel_t, rel_i, rel_o, r_pw)
        run(wgt_t, wgt_i, wgt_o, r_pw)
        ib = wid * i_pw
        pltpu.sync_copy(item_i.at[pl.ds(ib, i_pw)], idx_v.at[pl.ds(0, i_pw)])
        pltpu.async_copy(ent_t.at[idx_v.at[pl.ds(0, i_pw)]],
                         rows_v.at[pl.ds(0, i_pw)], gsem).wait()
        pltpu.sync_copy(rows_v.at[pl.ds(0, i_pw)], item_o.at[pl.ds(ib, i_pw)])

    return gather_kernel


def _attention_body(m_triples, tb, ent, rel, wgt, itemr,
                    w1, w2, w3, aw1, aw2, aw3, out):
    rt = tb * m_triples
    f32 = jnp.float32
    hi = jax.lax.Precision.HIGHEST
    dim = itemr.shape[-1]

    col = lax.broadcasted_iota(jnp.int32, (tb, rt), 1) // m_triples
    row = lax.broadcasted_iota(jnp.int32, (tb, rt), 0)
    s_t = jnp.where(col == row, f32(1), f32(0))

    def mm(a, b):
        return jnp.dot(a, b, precision=hi, preferred_element_type=f32)

    def attn(h, r, t, u1, u2, u3, mul_ht):
        x = h * t if mul_ht else h
        u = jnp.maximum(mm(x, u1[:dim]) + mm(r, u1[dim:]), 0.0)
        v = jnp.maximum(mm(u, u2), 0.0)
        l = jax.nn.sigmoid(mm(v, u3))
        e = jnp.exp(l)
        num = mm(s_t, e * t)
        den = mm(s_t, e)
        return num / den

    w1v, w2v, w3v = w1[...], w2[...], w3[...]
    aw1v, aw2v, aw3v = aw1[...], aw2[...], aw3[...]

    e_u = mm(s_t, ent[0]) * (1.0 / m_triples)
    for k in range(2):
        e_u = e_u + attn(ent[k], rel[k], ent[2 + k], w1v, w2v, w3v, False)
    for k in range(2):
        e_u = e_u + attn(ent[4 + k], wgt[k], ent[6 + k], aw1v, aw2v, aw3v, True)

    e_v = itemr[...]
    for k in range(2):
        e_v = e_v + attn(ent[8 + k], rel[2 + k], ent[10 + k], w1v, w2v, w3v, False)
    for k in range(2):
        e_v = e_v + attn(ent[12 + k], wgt[2 + k], ent[14 + k], aw1v, aw2v, aw3v, True)

    s = jnp.sum(e_u * e_v, axis=1, keepdims=True)
    out[0] = jax.nn.sigmoid(s)


def _tc_attention(ent3, rel3, wgt3, item_rows, w1, w2, w3, aw1, aw2, aw3,
                  batch, m_triples, tb):
    rt = tb * m_triples
    dim = item_rows.shape[-1]
    n_tiles = batch // tb
    grid = (n_tiles,)
    body = functools.partial(_attention_body, m_triples, tb)
    return pl.pallas_call(
        body,
        grid=grid,
        in_specs=[
            pl.BlockSpec((16, rt, dim), lambda i: (0, i, 0)),
            pl.BlockSpec((4, rt, dim), lambda i: (0, i, 0)),
            pl.BlockSpec((4, rt, dim), lambda i: (0, i, 0)),
            pl.BlockSpec((tb, dim), lambda i: (i, 0)),
            pl.BlockSpec((2 * dim, dim), lambda i: (0, 0)),
            pl.BlockSpec((dim, dim), lambda i: (0, 0)),
            pl.BlockSpec((dim, 1), lambda i: (0, 0)),
            pl.BlockSpec((2 * dim, dim), lambda i: (0, 0)),
            pl.BlockSpec((dim, dim), lambda i: (0, 0)),
            pl.BlockSpec((dim, 1), lambda i: (0, 0)),
        ],
        out_specs=pl.BlockSpec((1, tb, 1), lambda i: (i, 0, 0)),
        out_shape=jax.ShapeDtypeStruct((n_tiles, tb, 1), jnp.float32),
    )(ent3, rel3, wgt3, item_rows, w1, w2, w3, aw1, aw2, aw3)


def kernel(items, user_triple_set, user_triple_set_e, item_triple_set,
           item_triple_set_i, entity_table, relation_table, weight_table,
           att_w1, att_w2, att_w3, atta_w1, atta_w2, atta_w3):
    i32 = jnp.int32
    n_layer = user_triple_set.shape[1]
    batch = user_triple_set.shape[2]
    m_triples = user_triple_set.shape[3]
    dim = entity_table.shape[1]
    bm = batch * m_triples

    def flat(x):
        return x.reshape(-1)

    ent_segments = []
    for ts in (user_triple_set, user_triple_set_e,
               item_triple_set, item_triple_set_i):
        for pos in (0, 2):
            for layer in range(n_layer):
                ent_segments.append(flat(ts[pos, layer]))
    ent_idx = jnp.concatenate(ent_segments).astype(i32)

    rel_idx = jnp.concatenate(
        [flat(user_triple_set[1, layer]) for layer in range(n_layer)]
        + [flat(item_triple_set[1, layer]) for layer in range(n_layer)]
    ).astype(i32)
    wgt_idx = jnp.concatenate(
        [flat(user_triple_set_e[1, layer]) for layer in range(n_layer)]
        + [flat(item_triple_set_i[1, layer]) for layer in range(n_layer)]
    ).astype(i32)
    item_idx = items.astype(i32)

    n_ent = ent_idx.shape[0]
    n_rel = rel_idx.shape[0]
    gather = _make_sc_gather(n_ent, n_rel, batch, dim)
    ent_rows, rel_rows, wgt_rows, item_rows = gather(
        entity_table, relation_table, weight_table,
        ent_idx, rel_idx, wgt_idx, item_idx)

    ent3 = ent_rows.reshape(16, bm, dim)
    rel3 = rel_rows.reshape(4, bm, dim)
    wgt3 = wgt_rows.reshape(4, bm, dim)

    tb = 32
    scores3 = _tc_attention(ent3, rel3, wgt3, item_rows,
                            att_w1, att_w2, att_w3,
                            atta_w1, atta_w2, atta_w3,
                            batch, m_triples, tb)
    return scores3.reshape(batch)


# speedup vs baseline: 3.7055x; 2.1525x over previous
import functools

import jax
import jax.numpy as jnp
from jax import lax
from jax.experimental import pallas as pl
from jax.experimental.pallas import tpu as pltpu
from jax.experimental.pallas import tpu_sc as plsc

_CH = 640


def _make_sc_gather(n_ent, n_rel, n_item, dim):
    info = plsc.get_sparse_core_info()
    _NC = info.num_cores
    _NS = info.num_subcores
    _NW = _NC * _NS
    e_pw = n_ent // _NW
    r_pw = n_rel // _NW
    i_pw = n_item // _NW
    mesh = plsc.VectorSubcoreMesh(core_axis_name="c", subcore_axis_name="s")
    f32 = jnp.float32

    @functools.partial(
        pl.kernel,
        mesh=mesh,
        compiler_params=pltpu.CompilerParams(use_tc_tiling_on_sc=False),
        out_type=[
            jax.ShapeDtypeStruct((n_ent, dim), f32),
            jax.ShapeDtypeStruct((n_rel, dim), f32),
            jax.ShapeDtypeStruct((n_rel, dim), f32),
            jax.ShapeDtypeStruct((n_item, dim), f32),
        ],
        scratch_types=[
            pltpu.VMEM((e_pw,), jnp.int32),
            pltpu.VMEM((_CH, dim), f32),
            pltpu.SemaphoreType.DMA,
        ],
    )
    def gather_kernel(ent_t, rel_t, wgt_t, ent_i, rel_i, wgt_i, item_i,
                      ent_o, rel_o, wgt_o, item_o, idx_v, rows_v, gsem):
        wid = lax.axis_index("s") * _NC + lax.axis_index("c")

        def run(table, ihbm, ohbm, n_per_w):
            base = wid * n_per_w
            pltpu.sync_copy(ihbm.at[pl.ds(base, n_per_w)],
                            idx_v.at[pl.ds(0, n_per_w)])

            def body(c, carry):
                off = c * _CH
                pltpu.async_copy(table.at[idx_v.at[pl.ds(off, _CH)]],
                                 rows_v, gsem).wait()
                pltpu.sync_copy(rows_v, ohbm.at[pl.ds(base + off, _CH)])
                return carry

            lax.fori_loop(0, n_per_w // _CH, body, 0)

        run(ent_t, ent_i, ent_o, e_pw)
        run(rel_t, rel_i, rel_o, r_pw)
        run(wgt_t, wgt_i, wgt_o, r_pw)
        ib = wid * i_pw
        pltpu.sync_copy(item_i.at[pl.ds(ib, i_pw)], idx_v.at[pl.ds(0, i_pw)])
        pltpu.async_copy(ent_t.at[idx_v.at[pl.ds(0, i_pw)]],
                         rows_v.at[pl.ds(0, i_pw)], gsem).wait()
        pltpu.sync_copy(rows_v.at[pl.ds(0, i_pw)], item_o.at[pl.ds(ib, i_pw)])

    return gather_kernel


def _attention_body(m_triples, tb, ent, rel, wgt, itemr,
                    w1, w2, w3, aw1, aw2, aw3, out):
    rt = tb * m_triples
    n_stk = 4 * rt
    f32 = jnp.float32
    bf16 = jnp.bfloat16
    dim = itemr.shape[-1]

    col = lax.broadcasted_iota(jnp.int32, (4 * tb, n_stk), 1) // m_triples
    row = lax.broadcasted_iota(jnp.int32, (4 * tb, n_stk), 0)
    s_t = jnp.where(col == row, f32(1), f32(0))

    def mm(a, b):
        return jnp.dot(a, b, preferred_element_type=f32)

    def mlp(x, r, u1, u2, u3):
        xr = jnp.concatenate([x, r], axis=1).astype(bf16)
        u = jnp.maximum(mm(xr, u1.astype(bf16)), 0.0).astype(bf16)
        v = jnp.maximum(mm(u, u2.astype(bf16)), 0.0).astype(bf16)
        return jax.nn.sigmoid(mm(v, u3.astype(bf16)))

    def branch(h4, r4, t4, u1, u2, u3, mul_ht):
        x = h4 * t4 if mul_ht else h4
        l = mlp(x, r4, u1, u2, u3)
        e = jnp.exp(l)
        ett = jnp.concatenate([e * t4, e], axis=1)
        nd = mm(s_t, ett)
        return nd[:, :dim] / nd[:, dim:]

    def collapse(a, b):
        return ent[a:b].reshape(n_stk, dim)

    att_s = branch(collapse(0, 4), rel[...].reshape(n_stk, dim),
                   collapse(4, 8), w1[...], w2[...], w3[...], False)
    att_a = branch(collapse(8, 12), wgt[...].reshape(n_stk, dim),
                   collapse(12, 16), aw1[...], aw2[...], aw3[...], True)

    base_u = mm(s_t[:tb, :rt], ent[0]) * (1.0 / m_triples)
    e_u = base_u + att_s[:tb] + att_s[tb:2 * tb] + att_a[:tb] + att_a[tb:2 * tb]
    e_v = (itemr[...] + att_s[2 * tb:3 * tb] + att_s[3 * tb:]
           + att_a[2 * tb:3 * tb] + att_a[3 * tb:])

    s = jnp.sum(e_u * e_v, axis=1, keepdims=True)
    out[0] = jax.nn.sigmoid(s)


def _tc_attention(ent3, rel3, wgt3, item_rows, w1, w2, w3, aw1, aw2, aw3,
                  batch, m_triples, tb):
    rt = tb * m_triples
    dim = item_rows.shape[-1]
    n_tiles = batch // tb
    grid = (n_tiles,)
    body = functools.partial(_attention_body, m_triples, tb)
    return pl.pallas_call(
        body,
        grid=grid,
        in_specs=[
            pl.BlockSpec((16, rt, dim), lambda i: (0, i, 0)),
            pl.BlockSpec((4, rt, dim), lambda i: (0, i, 0)),
            pl.BlockSpec((4, rt, dim), lambda i: (0, i, 0)),
            pl.BlockSpec((tb, dim), lambda i: (i, 0)),
            pl.BlockSpec((2 * dim, dim), lambda i: (0, 0)),
            pl.BlockSpec((dim, dim), lambda i: (0, 0)),
            pl.BlockSpec((dim, 1), lambda i: (0, 0)),
            pl.BlockSpec((2 * dim, dim), lambda i: (0, 0)),
            pl.BlockSpec((dim, dim), lambda i: (0, 0)),
            pl.BlockSpec((dim, 1), lambda i: (0, 0)),
        ],
        out_specs=pl.BlockSpec((1, tb, 1), lambda i: (i, 0, 0)),
        out_shape=jax.ShapeDtypeStruct((n_tiles, tb, 1), jnp.float32),
    )(ent3, rel3, wgt3, item_rows, w1, w2, w3, aw1, aw2, aw3)


def kernel(items, user_triple_set, user_triple_set_e, item_triple_set,
           item_triple_set_i, entity_table, relation_table, weight_table,
           att_w1, att_w2, att_w3, atta_w1, atta_w2, atta_w3):
    i32 = jnp.int32
    n_layer = user_triple_set.shape[1]
    batch = user_triple_set.shape[2]
    m_triples = user_triple_set.shape[3]
    dim = entity_table.shape[1]
    bm = batch * m_triples

    def flat(x):
        return x.reshape(-1)

    ent_segments = []
    for pair in ((user_triple_set, item_triple_set),
                 (user_triple_set_e, item_triple_set_i)):
        for pos in (0, 2):
            for ts in pair:
                for layer in range(n_layer):
                    ent_segments.append(flat(ts[pos, layer]))
    ent_idx = jnp.concatenate(ent_segments).astype(i32)

    rel_idx = jnp.concatenate(
        [flat(user_triple_set[1, layer]) for layer in range(n_layer)]
        + [flat(item_triple_set[1, layer]) for layer in range(n_layer)]
    ).astype(i32)
    wgt_idx = jnp.concatenate(
        [flat(user_triple_set_e[1, layer]) for layer in range(n_layer)]
        + [flat(item_triple_set_i[1, layer]) for layer in range(n_layer)]
    ).astype(i32)
    item_idx = items.astype(i32)

    n_ent = ent_idx.shape[0]
    n_rel = rel_idx.shape[0]
    gather = _make_sc_gather(n_ent, n_rel, batch, dim)
    ent_rows, rel_rows, wgt_rows, item_rows = gather(
        entity_table, relation_table, weight_table,
        ent_idx, rel_idx, wgt_idx, item_idx)

    ent3 = ent_rows.reshape(16, bm, dim)
    rel3 = rel_rows.reshape(4, bm, dim)
    wgt3 = wgt_rows.reshape(4, bm, dim)

    tb = 32
    scores3 = _tc_attention(ent3, rel3, wgt3, item_rows,
                            att_w1, att_w2, att_w3,
                            atta_w1, atta_w2, atta_w3,
                            batch, m_triples, tb)
    return scores3.reshape(batch)


# speedup vs baseline: 3.8223x; 1.0315x over previous
import functools

import jax
import jax.numpy as jnp
from jax import lax
from jax.experimental import pallas as pl
from jax.experimental.pallas import tpu as pltpu
from jax.experimental.pallas import tpu_sc as plsc

_CH = 640


def _make_sc_gather(n_ent, n_rel, n_item, dim):
    info = plsc.get_sparse_core_info()
    _NC = info.num_cores
    _NS = info.num_subcores
    _NW = _NC * _NS
    e_pw = n_ent // _NW
    r_pw = n_rel // _NW
    i_pw = n_item // _NW
    mesh = plsc.VectorSubcoreMesh(core_axis_name="c", subcore_axis_name="s")
    f32 = jnp.float32

    @functools.partial(
        pl.kernel,
        mesh=mesh,
        compiler_params=pltpu.CompilerParams(use_tc_tiling_on_sc=False),
        out_type=[
            jax.ShapeDtypeStruct((n_ent, dim), f32),
            jax.ShapeDtypeStruct((n_rel, dim), f32),
            jax.ShapeDtypeStruct((n_rel, dim), f32),
            jax.ShapeDtypeStruct((n_item, dim), f32),
        ],
        scratch_types=[
            pltpu.VMEM((e_pw,), jnp.int32),
            pltpu.VMEM((2, _CH, dim), f32),
            pltpu.SemaphoreType.DMA,
            pltpu.SemaphoreType.DMA,
            pltpu.SemaphoreType.DMA,
            pltpu.SemaphoreType.DMA,
        ],
    )
    def gather_kernel(ent_t, rel_t, wgt_t, ent_i, rel_i, wgt_i, item_i,
                      ent_o, rel_o, wgt_o, item_o, idx_v, rows_v,
                      gsem0, gsem1, wsem0, wsem1):
        wid = lax.axis_index("s") * _NC + lax.axis_index("c")
        gsems = (gsem0, gsem1)
        wsems = (wsem0, wsem1)

        def run(table, ihbm, ohbm, n_per_w):
            base = wid * n_per_w
            nch = n_per_w // _CH
            pltpu.sync_copy(ihbm.at[pl.ds(base, n_per_w)],
                            idx_v.at[pl.ds(0, n_per_w)])

            def g_copy(c, b):
                return pltpu.make_async_copy(
                    table.at[idx_v.at[pl.ds(c * _CH, _CH)]],
                    rows_v.at[b], gsems[b])

            def w_copy(c, b):
                return pltpu.make_async_copy(
                    rows_v.at[b], ohbm.at[pl.ds(base + c * _CH, _CH)],
                    wsems[b])

            g_copy(0, 0).start()
            g_copy(1, 1).start()

            def body(p, carry):
                for b in range(2):
                    c = 2 * p + b
                    g_copy(c, b).wait()
                    w_copy(c, b).start()
                    w_copy(c, b).wait()

                    @pl.when(c + 2 < nch)
                    def _():
                        g_copy(c + 2, b).start()
                return carry

            lax.fori_loop(0, nch // 2, body, 0)

        run(ent_t, ent_i, ent_o, e_pw)
        run(rel_t, rel_i, rel_o, r_pw)
        run(wgt_t, wgt_i, wgt_o, r_pw)
        ib = wid * i_pw
        pltpu.sync_copy(item_i.at[pl.ds(ib, i_pw)], idx_v.at[pl.ds(0, i_pw)])
        pltpu.async_copy(ent_t.at[idx_v.at[pl.ds(0, i_pw)]],
                         rows_v.at[0, pl.ds(0, i_pw)], gsem0).wait()
        pltpu.sync_copy(rows_v.at[0, pl.ds(0, i_pw)],
                        item_o.at[pl.ds(ib, i_pw)])

    return gather_kernel


def _attention_body(m_triples, tb, ent, rel, wgt, itemr,
                    w1, w2, w3, aw1, aw2, aw3, out):
    rt = tb * m_triples
    n_stk = 4 * rt
    f32 = jnp.float32
    bf16 = jnp.bfloat16
    dim = itemr.shape[-1]

    col = lax.broadcasted_iota(jnp.int32, (4 * tb, n_stk), 1) // m_triples
    row = lax.broadcasted_iota(jnp.int32, (4 * tb, n_stk), 0)
    s_t = jnp.where(col == row, f32(1), f32(0))

    def mm(a, b):
        return jnp.dot(a, b, preferred_element_type=f32)

    def mlp(x, r, u1, u2, u3):
        xr = jnp.concatenate([x, r], axis=1).astype(bf16)
        u = jnp.maximum(mm(xr, u1.astype(bf16)), 0.0).astype(bf16)
        v = jnp.maximum(mm(u, u2.astype(bf16)), 0.0).astype(bf16)
        return jax.nn.sigmoid(mm(v, u3.astype(bf16)))

    def branch(h4, r4, t4, u1, u2, u3, mul_ht):
        x = h4 * t4 if mul_ht else h4
        l = mlp(x, r4, u1, u2, u3)
        e = jnp.exp(l)
        ett = jnp.concatenate([e * t4, e], axis=1)
        nd = mm(s_t, ett)
        return nd[:, :dim] / nd[:, dim:]

    def collapse(a, b):
        return ent[a:b].reshape(n_stk, dim)

    att_s = branch(collapse(0, 4), rel[...].reshape(n_stk, dim),
                   collapse(4, 8), w1[...], w2[...], w3[...], False)
    att_a = branch(collapse(8, 12), wgt[...].reshape(n_stk, dim),
                   collapse(12, 16), aw1[...], aw2[...], aw3[...], True)

    base_u = mm(s_t[:tb, :rt], ent[0]) * (1.0 / m_triples)
    e_u = base_u + att_s[:tb] + att_s[tb:2 * tb] + att_a[:tb] + att_a[tb:2 * tb]
    e_v = (itemr[...] + att_s[2 * tb:3 * tb] + att_s[3 * tb:]
           + att_a[2 * tb:3 * tb] + att_a[3 * tb:])

    s = jnp.sum(e_u * e_v, axis=1, keepdims=True)
    out[0] = jax.nn.sigmoid(s)


def _tc_attention(ent3, rel3, wgt3, item_rows, w1, w2, w3, aw1, aw2, aw3,
                  batch, m_triples, tb):
    rt = tb * m_triples
    dim = item_rows.shape[-1]
    n_tiles = batch // tb
    grid = (n_tiles,)
    body = functools.partial(_attention_body, m_triples, tb)
    return pl.pallas_call(
        body,
        grid=grid,
        in_specs=[
            pl.BlockSpec((16, rt, dim), lambda i: (0, i, 0)),
            pl.BlockSpec((4, rt, dim), lambda i: (0, i, 0)),
            pl.BlockSpec((4, rt, dim), lambda i: (0, i, 0)),
            pl.BlockSpec((tb, dim), lambda i: (i, 0)),
            pl.BlockSpec((2 * dim, dim), lambda i: (0, 0)),
            pl.BlockSpec((dim, dim), lambda i: (0, 0)),
            pl.BlockSpec((dim, 1), lambda i: (0, 0)),
            pl.BlockSpec((2 * dim, dim), lambda i: (0, 0)),
            pl.BlockSpec((dim, dim), lambda i: (0, 0)),
            pl.BlockSpec((dim, 1), lambda i: (0, 0)),
        ],
        out_specs=pl.BlockSpec((1, tb, 1), lambda i: (i, 0, 0)),
        out_shape=jax.ShapeDtypeStruct((n_tiles, tb, 1), jnp.float32),
    )(ent3, rel3, wgt3, item_rows, w1, w2, w3, aw1, aw2, aw3)


def kernel(items, user_triple_set, user_triple_set_e, item_triple_set,
           item_triple_set_i, entity_table, relation_table, weight_table,
           att_w1, att_w2, att_w3, atta_w1, atta_w2, atta_w3):
    i32 = jnp.int32
    n_layer = user_triple_set.shape[1]
    batch = user_triple_set.shape[2]
    m_triples = user_triple_set.shape[3]
    dim = entity_table.shape[1]
    bm = batch * m_triples

    def flat(x):
        return x.reshape(-1)

    ent_segments = []
    for pair in ((user_triple_set, item_triple_set),
                 (user_triple_set_e, item_triple_set_i)):
        for pos in (0, 2):
            for ts in pair:
                for layer in range(n_layer):
                    ent_segments.append(flat(ts[pos, layer]))
    ent_idx = jnp.concatenate(ent_segments).astype(i32)

    rel_idx = jnp.concatenate(
        [flat(user_triple_set[1, layer]) for layer in range(n_layer)]
        + [flat(item_triple_set[1, layer]) for layer in range(n_layer)]
    ).astype(i32)
    wgt_idx = jnp.concatenate(
        [flat(user_triple_set_e[1, layer]) for layer in range(n_layer)]
        + [flat(item_triple_set_i[1, layer]) for layer in range(n_layer)]
    ).astype(i32)
    item_idx = items.astype(i32)

    n_ent = ent_idx.shape[0]
    n_rel = rel_idx.shape[0]
    gather = _make_sc_gather(n_ent, n_rel, batch, dim)
    ent_rows, rel_rows, wgt_rows, item_rows = gather(
        entity_table, relation_table, weight_table,
        ent_idx, rel_idx, wgt_idx, item_idx)

    ent3 = ent_rows.reshape(16, bm, dim)
    rel3 = rel_rows.reshape(4, bm, dim)
    wgt3 = wgt_rows.reshape(4, bm, dim)

    tb = 32
    scores3 = _tc_attention(ent3, rel3, wgt3, item_rows,
                            att_w1, att_w2, att_w3,
                            atta_w1, atta_w2, atta_w3,
                            batch, m_triples, tb)
    return scores3.reshape(batch)
